# Initial kernel scaffold; baseline (speedup 1.0000x reference)
import jax
import jax.numpy as jnp
from jax.experimental import pallas as pl


def kernel(x, table):
    raise NotImplementedError("write your pallas kernel here")


# speedup vs baseline: 1.8693x; 1.8693x over previous
import functools

import jax
import jax.numpy as jnp
from jax import lax
from jax.experimental import pallas as pl
from jax.experimental.pallas import tpu as pltpu
from jax.experimental.pallas import tpu_sc as plsc

CHUNK = 128
K = 8


def _make_lookup(V, D, B, NC, NS):
    NW = NC * NS
    rows_per_w = B // NW
    cpw = rows_per_w // CHUNK
    G = cpw // K
    mesh = plsc.VectorSubcoreMesh(core_axis_name="c", subcore_axis_name="s")

    @functools.partial(
        pl.kernel,
        mesh=mesh,
        out_type=jax.ShapeDtypeStruct((B, D), jnp.float32),
        scratch_types=[
            pltpu.VMEM((cpw, CHUNK), jnp.int32),
            pltpu.VMEM((K, CHUNK, D), jnp.float32),
            pltpu.SemaphoreType.DMA,
            pltpu.SemaphoreType.DMA,
        ],
        compiler_params=pltpu.CompilerParams(use_tc_tiling_on_sc=False),
    )
    def lookup(table_hbm, x_hbm, out_hbm, idx_v, rows_v, gsem, ssem):
        wid = lax.axis_index("s") * NC + lax.axis_index("c")
        chunk0 = wid * cpw
        pltpu.sync_copy(x_hbm.at[pl.ds(chunk0, cpw)], idx_v)

        def group(g, carry):
            j0 = g * K
            gds = [
                pltpu.async_copy(table_hbm.at[idx_v.at[j0 + b]], rows_v.at[b], gsem)
                for b in range(K)
            ]
            for d in gds:
                d.wait()
            sds = [
                pltpu.async_copy(
                    rows_v.at[b],
                    out_hbm.at[pl.ds((chunk0 + j0 + b) * CHUNK, CHUNK)],
                    ssem,
                )
                for b in range(K)
            ]
            for d in sds:
                d.wait()
            return carry

        lax.fori_loop(0, G, group, 0)

    return lookup


def kernel(x, table):
    B, H = x.shape
    V, D = table.shape
    n = B * H
    x2d = x.astype(jnp.int32).reshape(n // CHUNK, CHUNK)
    info = plsc.get_sparse_core_info()
    out = _make_lookup(V, D, n, info.num_cores, info.num_subcores)(table, x2d)
    return out.reshape(B, H, D)
